# baseline (speedup 1.0000x reference)
import functools

import jax
import jax.numpy as jnp
from jax import lax
from jax.experimental import pallas as pl
from jax.experimental.pallas import tpu as pltpu
from jax.experimental.pallas import tpu_sc as plsc

_TEMP = 10.0
_NC = 2
_NS = 16
_CR = 8
_CC = 1024

_GDN = lax.GatherDimensionNumbers(
    offset_dims=(), collapsed_slice_dims=(0,), start_index_map=(0,)
)


def _lane_gather(vec, perm):
    return lax.gather(vec, perm[:, None], _GDN, (1,),
                      mode=lax.GatherScatterMode.PROMISE_IN_BOUNDS)


def _make_sc_kernel(R, M):
    Mc = M - 2
    NW = _NC * _NS
    rows_per_w = R // NW
    groups = rows_per_w // _CR
    halves = M // _CC
    nvec = _CC // 16

    mesh = plsc.VectorSubcoreMesh(core_axis_name="c", subcore_axis_name="s")

    @functools.partial(
        pl.kernel,
        out_type=jax.ShapeDtypeStruct((R, M), jnp.float32),
        mesh=mesh,
        compiler_params=pltpu.CompilerParams(use_tc_tiling_on_sc=True),
        scratch_types=[
            pltpu.VMEM((_CR, _CC), jnp.float32),
            pltpu.VMEM((_CR, _CC), jnp.float32),
            pltpu.VMEM((_CR, _CC), jnp.float32),
            pltpu.VMEM((_CR, _CC), jnp.float32),
            pltpu.VMEM((_CR, _CC), jnp.float32),
            pltpu.VMEM((_CR, _CC), jnp.float32),
            pltpu.VMEM((_CR, Mc), jnp.int32),
            pltpu.VMEM((_CR, Mc), jnp.int32),
            pltpu.VMEM((M,), jnp.float32),
            pltpu.VMEM((M,), jnp.float32),
            pltpu.SemaphoreType.DMA,
            pltpu.SemaphoreType.DMA,
            pltpu.SemaphoreType.DMA,
            pltpu.SemaphoreType.DMA,
            pltpu.SemaphoreType.DMA,
            pltpu.SemaphoreType.DMA,
        ],
    )
    def sc_kernel(x_hbm, o_hbm, t0_hbm, t1_hbm, ci_hbm, out_hbm,
                  x_a, o_a, out_a, x_b, o_b, out_b, ci_0, ci_1,
                  t0_v, t1_v, sin_a, sin_b, sci_0, sci_1, sout_a, sout_b):
        sides = ((x_a, o_a, out_a, sin_a, sout_a),
                 (x_b, o_b, out_b, sin_b, sout_b))
        cis = ((ci_0, sci_0), (ci_1, sci_1))
        wid = lax.axis_index("s") * _NC + lax.axis_index("c")
        row_base = wid * rows_per_w
        pltpu.sync_copy(t0_hbm, t0_v)
        pltpu.sync_copy(t1_hbm, t1_v)

        @plsc.parallel_loop(0, M, 16, unroll=8)
        def _scale(i):
            t0_v[pl.ds(i, 16)] = t0_v[pl.ds(i, 16)] * _TEMP
            t1_v[pl.ds(i, 16)] = t1_v[pl.ds(i, 16)] * _TEMP

        iota = lax.iota(jnp.int32, 16)
        perm_q = jnp.maximum(iota - 2, 0)
        perm_p = jnp.minimum(iota + 14, 15)

        def in_descs(rr, h, side):
            xbuf, obuf = sides[side][0], sides[side][1]
            sem = sides[side][3]
            row0 = row_base + rr * _CR
            return (
                (x_hbm.at[pl.ds(row0, _CR), pl.ds(h * _CC, _CC)], xbuf, sem),
                (o_hbm.at[pl.ds(row0, _CR), pl.ds(h * _CC, _CC)], obuf, sem),
            )

        def ci_desc(rr, par):
            cibuf, sem = cis[par]
            row0 = row_base + rr * _CR
            return (ci_hbm.at[pl.ds(row0, _CR), :], cibuf, sem)

        def out_desc(rr, h, side):
            outbuf, sem = sides[side][2], sides[side][4]
            row0 = row_base + rr * _CR
            return (outbuf, out_hbm.at[pl.ds(row0, _CR), pl.ds(h * _CC, _CC)],
                    sem)

        def start(desc):
            pltpu.async_copy(*desc)

        def wait(desc):
            pltpu.make_async_copy(*desc).wait()

        def compute(h, xbuf, obuf, outbuf, cibuf):
            def emit(s, off, c16, t0, t1):
                xv = xbuf[s, pl.ds(off, 16)]
                ov = obuf[s, pl.ds(off, 16)]
                cz = c16 == 0
                v = jnp.where(cz, xv, ov)
                th = jnp.where(cz, t0, t1)
                denom = 1.0 + jnp.exp(th - _TEMP * jnp.abs(v))
                outbuf[s, pl.ds(off, 16)] = v / denom

            def boundary(col, zero):
                t0 = t0_v[pl.ds(col, 16)]
                t1 = t1_v[pl.ds(col, 16)]
                for s in range(_CR):
                    qv = cibuf[s, pl.ds(col, 16)]
                    lo = _lane_gather(qv, perm_q)
                    if zero:
                        c16 = jnp.where(iota >= 2, lo, 0)
                    else:
                        pv = cibuf[s, pl.ds(col - 16, 16)]
                        c16 = jnp.where(iota >= 2, lo,
                                        _lane_gather(pv, perm_p))
                    emit(s, col - h * _CC, c16, t0, t1)

            boundary(h * _CC, h == 0)

            @plsc.parallel_loop(1, _CC // 128, 1)
            def _b(j):
                boundary(h * _CC + 128 * j, False)

            @plsc.parallel_loop(0, _CC // 128, 1)
            def _main(j):
                colj = h * _CC + 128 * j
                qprev = [cibuf[s, pl.ds(colj, 16)] for s in range(_CR)]
                for kk2 in range(1, 8):
                    moff = colj + 16 * kk2
                    t0d = t0_v[pl.ds(moff, 16)]
                    t1d = t1_v[pl.ds(moff, 16)]
                    for s in range(_CR):
                        q = cibuf[s, pl.ds(moff, 16)]
                        c16 = jnp.where(iota >= 2, _lane_gather(q, perm_q),
                                        _lane_gather(qprev[s], perm_p))
                        emit(s, moff - h * _CC, c16, t0d, t1d)
                        qprev[s] = q

        start(ci_desc(0, 0))
        for d in in_descs(0, 0, 0):
            start(d)

        subs = ((0, 0, 0), (0, 1, 1), (1, 0, 0), (1, 1, 1))

        def body(r2, carry):
            r = 2 * r2
            for i, (dr, h, side) in enumerate(subs):
                rr = r + dr
                par = dr
                xbuf, obuf, outbuf, _, _ = sides[side]
                cibuf = cis[par][0]
                for d in in_descs(rr, h, side):
                    wait(d)
                if h == 0:
                    wait(ci_desc(rr, par))

                    @pl.when(rr + 1 < groups)
                    def _():
                        start(ci_desc(rr + 1, 1 - par))

                if i < 3:
                    ndr, nh, nside = subs[i + 1]
                    for d in in_descs(r + ndr, nh, nside):
                        start(d)
                else:
                    @pl.when(rr + 1 < groups)
                    def _():
                        for d in in_descs(rr + 1, 0, 0):
                            start(d)

                if i < 2:
                    @pl.when(r2 > 0)
                    def _():
                        wait(out_desc(rr - 1, h, side))
                else:
                    wait(out_desc(rr - 1, h, side))

                compute(h, xbuf, obuf, outbuf, cibuf)
                start(out_desc(rr, h, side))
            return carry

        lax.fori_loop(0, groups // 2, body, 0)
        wait(out_desc(groups - 1, 0, 0))
        wait(out_desc(groups - 1, 1, 1))

    return sc_kernel


def kernel(x, other_inputs_0, theta0, theta1, choice_indices):
    B, S, M = x.shape
    R = B * S
    sc = _make_sc_kernel(R, M)
    out = sc(
        x.reshape(R, M),
        other_inputs_0.reshape(R, M),
        theta0,
        theta1,
        choice_indices.reshape(R, M - 2),
    )
    return out.reshape(B, S, M)


# speedup vs baseline: 1.1301x; 1.1301x over previous
import functools

import jax
import jax.numpy as jnp
from jax import lax
from jax.experimental import pallas as pl
from jax.experimental.pallas import tpu as pltpu
from jax.experimental.pallas import tpu_sc as plsc

_TEMP = 10.0
_NC = 2
_NS = 16
_CR = 8
_CC = 1024

_GDN = lax.GatherDimensionNumbers(
    offset_dims=(), collapsed_slice_dims=(0,), start_index_map=(0,)
)


def _lane_gather(vec, perm):
    return lax.gather(vec, perm[:, None], _GDN, (1,),
                      mode=lax.GatherScatterMode.PROMISE_IN_BOUNDS)


def _make_sc_kernel(B, S, M):
    R = B * S
    Mc = M - 2
    NW = _NC * _NS
    rows_per_w = R // NW
    groups = rows_per_w // _CR
    nvec = _CC // 16

    mesh = plsc.VectorSubcoreMesh(core_axis_name="c", subcore_axis_name="s")

    @functools.partial(
        pl.kernel,
        out_type=jax.ShapeDtypeStruct((R, M), jnp.float32),
        mesh=mesh,
        compiler_params=pltpu.CompilerParams(use_tc_tiling_on_sc=True),
        scratch_types=[
            pltpu.VMEM((_CR, _CC), jnp.float32),
            pltpu.VMEM((_CR, _CC), jnp.float32),
            pltpu.VMEM((_CR, _CC), jnp.float32),
            pltpu.VMEM((_CR, _CC), jnp.float32),
            pltpu.VMEM((_CR, _CC), jnp.float32),
            pltpu.VMEM((_CR, _CC), jnp.float32),
            pltpu.VMEM((_CR, Mc), jnp.int32),
            pltpu.VMEM((_CR, Mc), jnp.int32),
            pltpu.VMEM((M,), jnp.float32),
            pltpu.VMEM((M,), jnp.float32),
            pltpu.SemaphoreType.DMA,
            pltpu.SemaphoreType.DMA,
            pltpu.SemaphoreType.DMA,
            pltpu.SemaphoreType.DMA,
            pltpu.SemaphoreType.DMA,
            pltpu.SemaphoreType.DMA,
        ],
    )
    def sc_kernel(x_hbm, o_hbm, t0_hbm, t1_hbm, ci_hbm, out_hbm,
                  x_a, o_a, out_a, x_b, o_b, out_b, ci_0, ci_1,
                  t0_v, t1_v, sin_a, sin_b, sci_0, sci_1, sout_a, sout_b):
        sides = ((x_a, o_a, out_a, sin_a, sout_a),
                 (x_b, o_b, out_b, sin_b, sout_b))
        cis = ((ci_0, sci_0), (ci_1, sci_1))
        wid = lax.axis_index("s") * _NC + lax.axis_index("c")
        row_base = wid * rows_per_w
        pltpu.sync_copy(t0_hbm, t0_v)
        pltpu.sync_copy(t1_hbm, t1_v)

        @plsc.parallel_loop(0, M, 16, unroll=8)
        def _scale(i):
            t0_v[pl.ds(i, 16)] = t0_v[pl.ds(i, 16)] * _TEMP
            t1_v[pl.ds(i, 16)] = t1_v[pl.ds(i, 16)] * _TEMP

        iota = lax.iota(jnp.int32, 16)
        perm_rot = jnp.where(iota < 2, iota + 14, iota - 2)

        def in_descs(rr, h, side):
            xbuf, obuf = sides[side][0], sides[side][1]
            sem = sides[side][3]
            row0 = row_base + rr * _CR
            return (
                (x_hbm.at[pl.ds(row0, _CR), pl.ds(h * _CC, _CC)], xbuf, sem),
                (o_hbm.at[pl.ds(row0, _CR), pl.ds(h * _CC, _CC)], obuf, sem),
            )

        def ci_desc(rr, par):
            cibuf, sem = cis[par]
            row0 = row_base + rr * _CR
            return (ci_hbm.at[row0 // S, pl.ds(row0 % S, _CR), :], cibuf, sem)

        def out_desc(rr, h, side):
            outbuf, sem = sides[side][2], sides[side][4]
            row0 = row_base + rr * _CR
            return (outbuf, out_hbm.at[pl.ds(row0, _CR), pl.ds(h * _CC, _CC)],
                    sem)

        def start(desc):
            pltpu.async_copy(*desc)

        def wait(desc):
            pltpu.make_async_copy(*desc).wait()

        def compute(h, xbuf, obuf, outbuf, cibuf):
            def emit(s, off, c16, t0, t1):
                xv = xbuf[s, pl.ds(off, 16)]
                ov = obuf[s, pl.ds(off, 16)]
                cz = c16 == 0
                v = jnp.where(cz, xv, ov)
                th = jnp.where(cz, t0, t1)
                denom = 1.0 + jnp.exp(th - _TEMP * jnp.abs(v))
                outbuf[s, pl.ds(off, 16)] = v / denom

            if h == 0:
                t0 = t0_v[pl.ds(0, 16)]
                t1 = t1_v[pl.ds(0, 16)]
                for s in range(_CR):
                    c16 = jnp.where(
                        iota >= 2,
                        _lane_gather(cibuf[s, pl.ds(0, 16)], perm_rot), 0)
                    emit(s, 0, c16, t0, t1)
            else:
                t0 = t0_v[pl.ds(M - 16, 16)]
                t1 = t1_v[pl.ds(M - 16, 16)]
                for s in range(_CR):
                    c16 = cibuf[s, pl.ds(Mc - 16, 16)]
                    emit(s, _CC - 16, c16, t0, t1)

            lo = h * nvec + (1 if h == 0 else 0)
            hi = (h + 1) * nvec - (0 if h == 0 else 1)

            @plsc.parallel_loop(lo, hi, 1, unroll=2)
            def _vec(kk):
                moff = kk * 16
                t0d = t0_v[pl.ds(moff, 16)]
                t1d = t1_v[pl.ds(moff, 16)]
                for s in range(_CR):
                    q = _lane_gather(cibuf[s, pl.ds(moff, 16)], perm_rot)
                    p = _lane_gather(cibuf[s, pl.ds(moff - 16, 16)], perm_rot)
                    c16 = jnp.where(iota >= 2, q, p)
                    emit(s, moff - h * _CC, c16, t0d, t1d)

        start(ci_desc(0, 0))
        for d in in_descs(0, 0, 0):
            start(d)

        subs = ((0, 0, 0), (0, 1, 1), (1, 0, 0), (1, 1, 1))

        def body(r2, carry):
            r = 2 * r2
            for i, (dr, h, side) in enumerate(subs):
                rr = r + dr
                par = dr
                xbuf, obuf, outbuf, _, _ = sides[side]
                cibuf = cis[par][0]
                for d in in_descs(rr, h, side):
                    wait(d)
                if h == 0:
                    wait(ci_desc(rr, par))

                    @pl.when(rr + 1 < groups)
                    def _():
                        start(ci_desc(rr + 1, 1 - par))

                if i < 3:
                    ndr, nh, nside = subs[i + 1]
                    for d in in_descs(r + ndr, nh, nside):
                        start(d)
                else:
                    @pl.when(rr + 1 < groups)
                    def _():
                        for d in in_descs(rr + 1, 0, 0):
                            start(d)

                if i < 2:
                    @pl.when(r2 > 0)
                    def _():
                        wait(out_desc(rr - 1, h, side))
                else:
                    wait(out_desc(rr - 1, h, side))

                compute(h, xbuf, obuf, outbuf, cibuf)
                start(out_desc(rr, h, side))
            return carry

        lax.fori_loop(0, groups // 2, body, 0)
        wait(out_desc(groups - 1, 0, 0))
        wait(out_desc(groups - 1, 1, 1))

    return sc_kernel


def kernel(x, other_inputs_0, theta0, theta1, choice_indices):
    B, S, M = x.shape
    R = B * S
    sc = _make_sc_kernel(B, S, M)
    out = sc(
        x.reshape(R, M),
        other_inputs_0.reshape(R, M),
        theta0,
        theta1,
        choice_indices,
    )
    return out.reshape(B, S, M)


# speedup vs baseline: 1.1848x; 1.0484x over previous
import functools

import jax
import jax.numpy as jnp
from jax import lax
from jax.experimental import pallas as pl
from jax.experimental.pallas import tpu as pltpu
from jax.experimental.pallas import tpu_sc as plsc

_TEMP = 10.0
_NC = 2
_NS = 16
_CR = 8
_CC = 1024


def _make_sc_kernel(R, M):
    NW = _NC * _NS
    rows_per_w = R // NW
    chunks = (rows_per_w // _CR) * (M // _CC)
    col_halves = M // _CC
    nvec = _CC // 16

    mesh = plsc.VectorSubcoreMesh(core_axis_name="c", subcore_axis_name="s")

    buf_types = [
        pltpu.VMEM((_CR, _CC), jnp.float32),
        pltpu.VMEM((_CR, _CC), jnp.float32),
        pltpu.VMEM((_CR, _CC), jnp.int32),
        pltpu.VMEM((_CR, _CC), jnp.float32),
    ]

    @functools.partial(
        pl.kernel,
        out_type=jax.ShapeDtypeStruct((R, M), jnp.float32),
        mesh=mesh,
        compiler_params=pltpu.CompilerParams(use_tc_tiling_on_sc=True),
        scratch_types=buf_types + buf_types + [
            pltpu.VMEM((M,), jnp.float32),
            pltpu.VMEM((M,), jnp.float32),
            pltpu.SemaphoreType.DMA,
            pltpu.SemaphoreType.DMA,
            pltpu.SemaphoreType.DMA,
            pltpu.SemaphoreType.DMA,
        ],
    )
    def sc_kernel(x_hbm, o_hbm, t0_hbm, t1_hbm, ci_hbm, out_hbm,
                  x_a, o_a, ci_a, out_a, x_b, o_b, ci_b, out_b,
                  t0_v, t1_v, sin_a, sin_b, sout_a, sout_b):
        bufs = ((x_a, o_a, ci_a, out_a, sin_a, sout_a),
                (x_b, o_b, ci_b, out_b, sin_b, sout_b))
        wid = lax.axis_index("s") * _NC + lax.axis_index("c")
        row_base = wid * rows_per_w
        pltpu.sync_copy(t0_hbm, t0_v)
        pltpu.sync_copy(t1_hbm, t1_v)

        @plsc.parallel_loop(0, M, 16, unroll=8)
        def _scale(i):
            t0_v[pl.ds(i, 16)] = t0_v[pl.ds(i, 16)] * _TEMP
            t1_v[pl.ds(i, 16)] = t1_v[pl.ds(i, 16)] * _TEMP

        def chunk_origin(g):
            row0 = row_base + (g // col_halves) * _CR
            cb = (g % col_halves) * _CC
            return row0, cb

        def in_descs(g, xbuf, obuf, cibuf, sem):
            row0, cb = chunk_origin(g)
            return (
                (x_hbm.at[pl.ds(row0, _CR), pl.ds(cb, _CC)], xbuf, sem),
                (o_hbm.at[pl.ds(row0, _CR), pl.ds(cb, _CC)], obuf, sem),
                (ci_hbm.at[pl.ds(row0, _CR), pl.ds(cb, _CC)], cibuf, sem),
            )

        def start_in(g, xbuf, obuf, cibuf, sem):
            for src, dst, s in in_descs(g, xbuf, obuf, cibuf, sem):
                pltpu.async_copy(src, dst, s)

        def wait_in(g, xbuf, obuf, cibuf, sem):
            for src, dst, s in in_descs(g, xbuf, obuf, cibuf, sem):
                pltpu.make_async_copy(src, dst, s).wait()

        def out_desc(g, outbuf, sem):
            row0, cb = chunk_origin(g)
            return (outbuf, out_hbm.at[pl.ds(row0, _CR), pl.ds(cb, _CC)], sem)

        def compute(g, xbuf, obuf, cibuf, outbuf):
            _, cb = chunk_origin(g)

            @plsc.parallel_loop(0, nvec, 1, unroll=4)
            def _vec(kk):
                moff = kk * 16
                t0 = t0_v[pl.ds(cb + moff, 16)]
                t1 = t1_v[pl.ds(cb + moff, 16)]
                for s in range(_CR):
                    c16 = cibuf[s, pl.ds(moff, 16)]
                    xv = xbuf[s, pl.ds(moff, 16)]
                    ov = obuf[s, pl.ds(moff, 16)]
                    cz = c16 == 0
                    v = jnp.where(cz, xv, ov)
                    th = jnp.where(cz, t0, t1)
                    denom = 1.0 + jnp.exp(th - _TEMP * jnp.abs(v))
                    outbuf[s, pl.ds(moff, 16)] = v / denom

        start_in(0, x_a, o_a, ci_a, sin_a)

        def body(g2, carry):
            for side in range(2):
                g = 2 * g2 + side
                xbuf, obuf, cibuf, outbuf, sin, sout = bufs[side]
                nxbuf, nobuf, ncibuf, _, nsin, _ = bufs[1 - side]
                wait_in(g, xbuf, obuf, cibuf, sin)

                @pl.when(g + 1 < chunks)
                def _():
                    start_in(g + 1, nxbuf, nobuf, ncibuf, nsin)

                @pl.when(g2 > 0)
                def _():
                    src, dst, s = out_desc(g - 2, outbuf, sout)
                    pltpu.make_async_copy(src, dst, s).wait()

                compute(g, xbuf, obuf, cibuf, outbuf)
                src, dst, s = out_desc(g, outbuf, sout)
                pltpu.async_copy(src, dst, s)
            return carry

        lax.fori_loop(0, chunks // 2, body, 0)
        pltpu.make_async_copy(*out_desc(chunks - 2, out_a, sout_a)).wait()
        pltpu.make_async_copy(*out_desc(chunks - 1, out_b, sout_b)).wait()

    return sc_kernel


def kernel(x, other_inputs_0, theta0, theta1, choice_indices):
    B, S, M = x.shape
    R = B * S
    cip = jnp.pad(choice_indices, ((0, 0), (0, 0), (2, 0)))
    sc = _make_sc_kernel(R, M)
    out = sc(
        x.reshape(R, M),
        other_inputs_0.reshape(R, M),
        theta0,
        theta1,
        cip.reshape(R, M),
    )
    return out.reshape(B, S, M)


# speedup vs baseline: 1.6405x; 1.3846x over previous
import functools

import jax
import jax.numpy as jnp
from jax import lax
from jax.experimental import pallas as pl
from jax.experimental.pallas import tpu as pltpu
from jax.experimental.pallas import tpu_sc as plsc

_TEMP = 10.0
_NC = 2
_NS = 16
_CR = 8
_CC = 1024


def _make_sc_kernel(R, M):
    NW = _NC * _NS
    rows_per_w = R // NW
    chunks = (rows_per_w // _CR) * (M // _CC)
    col_halves = M // _CC
    nvec = _CC // 16

    mesh = plsc.VectorSubcoreMesh(core_axis_name="c", subcore_axis_name="s")

    buf_types = [
        pltpu.VMEM((_CR, _CC), jnp.float32),
        pltpu.VMEM((_CR, _CC), jnp.float32),
        pltpu.VMEM((_CR, _CC), jnp.int32),
        pltpu.VMEM((_CR, _CC), jnp.float32),
    ]

    @functools.partial(
        pl.kernel,
        out_type=jax.ShapeDtypeStruct((R, M), jnp.float32),
        mesh=mesh,
        compiler_params=pltpu.CompilerParams(use_tc_tiling_on_sc=True),
        scratch_types=buf_types + buf_types + [
            pltpu.VMEM((M,), jnp.float32),
            pltpu.VMEM((M,), jnp.float32),
            pltpu.SemaphoreType.DMA,
            pltpu.SemaphoreType.DMA,
            pltpu.SemaphoreType.DMA,
            pltpu.SemaphoreType.DMA,
        ],
    )
    def sc_kernel(x_hbm, o_hbm, t0_hbm, t1_hbm, ci_hbm, out_hbm,
                  x_a, o_a, ci_a, out_a, x_b, o_b, ci_b, out_b,
                  t0_v, t1_v, sin_a, sin_b, sout_a, sout_b):
        bufs = ((x_a, o_a, ci_a, out_a, sin_a, sout_a),
                (x_b, o_b, ci_b, out_b, sin_b, sout_b))
        wid = lax.axis_index("s") * _NC + lax.axis_index("c")
        row_base = wid * rows_per_w
        pltpu.sync_copy(t0_hbm, t0_v)
        pltpu.sync_copy(t1_hbm, t1_v)

        @plsc.parallel_loop(0, M, 16, unroll=8)
        def _scale(i):
            t0_v[pl.ds(i, 16)] = t0_v[pl.ds(i, 16)] * _TEMP
            t1_v[pl.ds(i, 16)] = t1_v[pl.ds(i, 16)] * _TEMP

        def chunk_origin(g):
            row0 = row_base + (g // col_halves) * _CR
            cb = (g % col_halves) * _CC
            return row0, cb

        def in_descs(g, xbuf, obuf, cibuf, sem):
            row0, cb = chunk_origin(g)
            return (
                (x_hbm.at[pl.ds(row0, _CR), pl.ds(cb, _CC)], xbuf, sem),
                (o_hbm.at[pl.ds(row0, _CR), pl.ds(cb, _CC)], obuf, sem),
                (ci_hbm.at[pl.ds(row0, _CR), pl.ds(cb, _CC)], cibuf, sem),
            )

        def start_in(g, xbuf, obuf, cibuf, sem):
            for src, dst, s in in_descs(g, xbuf, obuf, cibuf, sem):
                pltpu.async_copy(src, dst, s)

        def wait_in(g, xbuf, obuf, cibuf, sem):
            for src, dst, s in in_descs(g, xbuf, obuf, cibuf, sem):
                pltpu.make_async_copy(src, dst, s).wait()

        def out_desc(g, outbuf, sem):
            row0, cb = chunk_origin(g)
            return (outbuf, out_hbm.at[pl.ds(row0, _CR), pl.ds(cb, _CC)], sem)

        def compute(g, xbuf, obuf, cibuf, outbuf):
            _, cb = chunk_origin(g)

            @plsc.parallel_loop(0, nvec, 1, unroll=2)
            def _vec(kk):
                moff = kk * 16
                t0 = t0_v[pl.ds(cb + moff, 16)]
                t1 = t1_v[pl.ds(cb + moff, 16)]
                for s in range(_CR):
                    c16 = cibuf[s, pl.ds(moff, 16)]
                    xv = xbuf[s, pl.ds(moff, 16)]
                    ov = obuf[s, pl.ds(moff, 16)]
                    cz = c16 == 0
                    v = jnp.where(cz, xv, ov)
                    th = jnp.where(cz, t0, t1)
                    denom = 1.0 + jnp.exp(th - _TEMP * jnp.abs(v))
                    outbuf[s, pl.ds(moff, 16)] = v / denom

        start_in(0, x_a, o_a, ci_a, sin_a)

        def body(g2, carry):
            for side in range(2):
                g = 2 * g2 + side
                xbuf, obuf, cibuf, outbuf, sin, sout = bufs[side]
                nxbuf, nobuf, ncibuf, _, nsin, _ = bufs[1 - side]
                wait_in(g, xbuf, obuf, cibuf, sin)

                @pl.when(g + 1 < chunks)
                def _():
                    start_in(g + 1, nxbuf, nobuf, ncibuf, nsin)

                @pl.when(g2 > 0)
                def _():
                    src, dst, s = out_desc(g - 2, outbuf, sout)
                    pltpu.make_async_copy(src, dst, s).wait()

                compute(g, xbuf, obuf, cibuf, outbuf)
                src, dst, s = out_desc(g, outbuf, sout)
                pltpu.async_copy(src, dst, s)
            return carry

        lax.fori_loop(0, chunks // 2, body, 0)
        pltpu.make_async_copy(*out_desc(chunks - 2, out_a, sout_a)).wait()
        pltpu.make_async_copy(*out_desc(chunks - 1, out_b, sout_b)).wait()

    return sc_kernel


def kernel(x, other_inputs_0, theta0, theta1, choice_indices):
    B, S, M = x.shape
    R = B * S
    cip = jnp.pad(choice_indices, ((0, 0), (0, 0), (2, 0)))
    sc = _make_sc_kernel(R, M)
    out = sc(
        x.reshape(R, M),
        other_inputs_0.reshape(R, M),
        theta0,
        theta1,
        cip.reshape(R, M),
    )
    return out.reshape(B, S, M)
